# Initial kernel scaffold; baseline (speedup 1.0000x reference)
import jax
import jax.numpy as jnp
from jax.experimental import pallas as pl


def kernel(expression):
    raise NotImplementedError("write your pallas kernel here")


# speedup vs baseline: 1.0921x; 1.0921x over previous
import functools

import jax
import jax.numpy as jnp
from jax import lax
from jax.experimental import pallas as pl
from jax.experimental.pallas import tpu as pltpu
from jax.experimental.pallas import tpu_sc as plsc

N = 4096
G = 2048
NCOMP = 50
KNN = 15
KPAD = 16
NITER = 100


_PM = 256
_PCOLS = 64


def _proj_body(x_ref, v_ref, o_ref):
    o_ref[...] = lax.dot_general(
        x_ref[...], v_ref[...], (((1,), (0,)), ((), ())),
        preferred_element_type=jnp.float32,
        precision=lax.Precision.HIGHEST)


def _project(centered, v_pad):
    return pl.pallas_call(
        _proj_body,
        grid=(N // _PM,),
        in_specs=[
            pl.BlockSpec((_PM, G), lambda i: (i, 0)),
            pl.BlockSpec((G, _PCOLS), lambda i: (0, 0)),
        ],
        out_specs=pl.BlockSpec((_PM, _PCOLS), lambda i: (i, 0)),
        out_shape=jax.ShapeDtypeStruct((N, _PCOLS), jnp.float32),
        compiler_params=pltpu.CompilerParams(
            dimension_semantics=("parallel",)),
    )(centered, v_pad)


_BR = 256
_BIG = 3.4e38


def _knn_body(x_ref, t_ref, i_ref, d_ref, w_ref):
    blk = pl.program_id(0)
    x = x_ref[...]
    xb = x_ref[pl.ds(blk * _BR, _BR), :]
    g = lax.dot_general(xb, x, (((1,), (1,)), ((), ())),
                        preferred_element_type=jnp.float32,
                        precision=lax.Precision.HIGHEST)
    ones = jnp.ones((1, _PCOLS), jnp.float32)
    rsq_j = lax.dot_general(ones, x * x, (((1,), (1,)), ((), ())),
                            preferred_element_type=jnp.float32,
                            precision=lax.Precision.HIGHEST)
    rsq_i = jnp.sum(xb * xb, axis=1, keepdims=True)
    sqd = jnp.maximum(rsq_i + rsq_j - 2.0 * g, 0.0)

    colidx = lax.broadcasted_iota(jnp.int32, (_BR, N), 1)
    work = sqd
    mask = jnp.zeros((_BR, N), jnp.bool_)
    for k in range(KNN):
        mv = jnp.min(work, axis=1, keepdims=True)
        cand = jnp.where(work == mv, colidx, N)
        sel = jnp.min(cand, axis=1, keepdims=True)
        onehot = colidx == sel
        mask = jnp.logical_or(mask, onehot)
        i_ref[:, k:k + 1] = sel
        d_ref[:, k:k + 1] = mv
        work = jnp.where(onehot, _BIG, work)
    i_ref[:, KNN:KPAD] = jnp.zeros((_BR, 1), jnp.int32)
    d_ref[:, KNN:KPAD] = jnp.full((_BR, 1), jnp.inf, jnp.float32)

    e = jnp.where(mask, jnp.exp(sqd * -0.5), 0.0)
    tsum = jnp.sum(e, axis=1, keepdims=True)
    t_ref[...] = e / tsum

    dv = d_ref[...]
    e2 = jnp.exp(-dv)
    dsum = jnp.sum(e2, axis=1, keepdims=True)
    w_ref[...] = e2 / dsum


def _knn_transition(pca_pad):
    return pl.pallas_call(
        _knn_body,
        grid=(N // _BR,),
        in_specs=[pl.BlockSpec((N, _PCOLS), lambda i: (0, 0))],
        out_specs=[
            pl.BlockSpec((_BR, N), lambda i: (i, 0)),
            pl.BlockSpec((_BR, KPAD), lambda i: (i, 0)),
            pl.BlockSpec((_BR, KPAD), lambda i: (i, 0)),
            pl.BlockSpec((_BR, KPAD), lambda i: (i, 0)),
        ],
        out_shape=[
            jax.ShapeDtypeStruct((N, N), jnp.float32),
            jax.ShapeDtypeStruct((N, KPAD), jnp.int32),
            jax.ShapeDtypeStruct((N, KPAD), jnp.float32),
            jax.ShapeDtypeStruct((N, KPAD), jnp.float32),
        ],
        compiler_params=pltpu.CompilerParams(
            dimension_semantics=("arbitrary",)),
    )(pca_pad)


_RPT = N // 16
_LANES = 16


def _power_body(idx_hbm, w_hbm, d0_hbm, out_hbm, idx_v, w_v, d_v, dn_v, dsh):
    c = lax.axis_index("c")
    s = lax.axis_index("s")
    base = s * _RPT
    pltpu.sync_copy(idx_hbm.at[s], idx_v)
    pltpu.sync_copy(w_hbm.at[s], w_v)
    pltpu.sync_copy(d0_hbm, d_v)

    def step(_, carry):
        for r in range(_RPT // _LANES):
            lo = r * _LANES
            acc = jnp.zeros((_LANES,), jnp.float32)
            for k in range(KPAD):
                ii = idx_v[k, pl.ds(lo, _LANES)]
                acc = acc + w_v[k, pl.ds(lo, _LANES)] * plsc.load_gather(
                    d_v, [ii])
            dn_v[pl.ds(lo, _LANES)] = acc
        pltpu.sync_copy(dn_v, dsh.at[pl.ds(base, _RPT)])
        plsc.subcore_barrier()
        pltpu.sync_copy(dsh, d_v)
        plsc.subcore_barrier()
        return carry

    lax.fori_loop(0, NITER, step, 0)

    mx = jnp.full((_LANES,), -jnp.inf, jnp.float32)
    for j in range(N // _LANES):
        mx = jnp.maximum(mx, d_v[pl.ds(j * _LANES, _LANES)])
    m = lax.reduce_max(mx, (0,))
    for r in range(_RPT // _LANES):
        lo = r * _LANES
        dn_v[pl.ds(lo, _LANES)] = d_v[pl.ds(base + lo, _LANES)] / m

    @pl.when(c == 0)
    def _write():
        pltpu.sync_copy(dn_v, out_hbm.at[pl.ds(base, _RPT)])


def _power_iteration(idx_sc, w_sc, d0):
    mesh = plsc.VectorSubcoreMesh(core_axis_name="c", subcore_axis_name="s")
    kern = functools.partial(
        pl.kernel,
        out_type=jax.ShapeDtypeStruct((N,), jnp.float32),
        mesh=mesh,
        scratch_types=[
            pltpu.VMEM((KPAD, _RPT), jnp.int32),
            pltpu.VMEM((KPAD, _RPT), jnp.float32),
            pltpu.VMEM((N,), jnp.float32),
            pltpu.VMEM((_RPT,), jnp.float32),
            pltpu.VMEM_SHARED((N,), jnp.float32),
        ],
        compiler_params=pltpu.CompilerParams(needs_layout_passes=False),
    )(_power_body)
    return kern(idx_sc, w_sc, d0)


def kernel(expression):
    data = expression
    data_mean = jnp.mean(data, axis=0)
    centered = data - data_mean

    cov = centered.T @ centered / (N - 1)
    eigenvalues, eigenvectors = jnp.linalg.eigh(cov)
    order = jnp.argsort(eigenvalues)[::-1]
    v = eigenvectors[:, order][:, :NCOMP]
    v_pad = jnp.pad(v, ((0, 0), (0, _PCOLS - NCOMP)))

    pca_pad = _project(centered, v_pad)
    pca_data = pca_pad[:, :NCOMP]

    transition, knn_idx, _, w = _knn_transition(pca_pad)

    idx_sc = knn_idx.reshape(16, _RPT, KPAD).transpose(0, 2, 1)
    w_sc = w.reshape(16, _RPT, KPAD).transpose(0, 2, 1)
    d0 = jnp.zeros((N,), jnp.float32).at[0].set(1.0)
    pseudotime = _power_iteration(idx_sc, w_sc, d0)

    return (pseudotime, transition, pca_data)


# speedup vs baseline: 1.0932x; 1.0011x over previous
import functools

import jax
import jax.numpy as jnp
from jax import lax
from jax.experimental import pallas as pl
from jax.experimental.pallas import tpu as pltpu
from jax.experimental.pallas import tpu_sc as plsc

N = 4096
G = 2048
NCOMP = 50
KNN = 15
KPAD = 16
NITER = 100


_PM = 256
_PCOLS = 64


def _proj_body(x_ref, v_ref, o_ref):
    o_ref[...] = lax.dot_general(
        x_ref[...], v_ref[...], (((1,), (0,)), ((), ())),
        preferred_element_type=jnp.float32,
        precision=lax.Precision.HIGHEST)


def _project(centered, v_pad):
    return pl.pallas_call(
        _proj_body,
        grid=(N // _PM,),
        in_specs=[
            pl.BlockSpec((_PM, G), lambda i: (i, 0)),
            pl.BlockSpec((G, _PCOLS), lambda i: (0, 0)),
        ],
        out_specs=pl.BlockSpec((_PM, _PCOLS), lambda i: (i, 0)),
        out_shape=jax.ShapeDtypeStruct((N, _PCOLS), jnp.float32),
        compiler_params=pltpu.CompilerParams(
            dimension_semantics=("parallel",)),
    )(centered, v_pad)


_BR = 128
_BIG = 3.4e38


def _knn_body(x_ref, t_ref, i_ref, w_ref):
    blk = pl.program_id(0)
    x = x_ref[...]
    xb = x_ref[pl.ds(blk * _BR, _BR), :]
    g = lax.dot_general(xb, x, (((1,), (1,)), ((), ())),
                        preferred_element_type=jnp.float32,
                        precision=lax.Precision.HIGHEST)
    ones = jnp.ones((1, _PCOLS), jnp.float32)
    rsq_j = lax.dot_general(ones, x * x, (((1,), (1,)), ((), ())),
                            preferred_element_type=jnp.float32,
                            precision=lax.Precision.HIGHEST)
    rsq_i = jnp.sum(xb * xb, axis=1, keepdims=True)
    sqd = jnp.maximum(rsq_i + rsq_j - 2.0 * g, 0.0)

    colidx = lax.broadcasted_iota(jnp.int32, (_BR, N), 1)
    bits = lax.bitcast_convert_type(sqd, jnp.int32)
    key = jnp.bitwise_or(jnp.bitwise_and(bits, jnp.int32(-4096)), colidx)
    work = key
    mask = jnp.zeros((_BR, N), jnp.bool_)
    dvals = []
    for k in range(KNN):
        mk = jnp.min(work, axis=1, keepdims=True)
        onehot = work == mk
        mask = jnp.logical_or(mask, onehot)
        work = jnp.where(onehot, jnp.int32(0x7FFFFFFF), work)
        i_ref[:, k:k + 1] = jnp.bitwise_and(mk, jnp.int32(4095))
        dvals.append(lax.bitcast_convert_type(
            jnp.bitwise_and(mk, jnp.int32(-4096)), jnp.float32))
    i_ref[:, KNN:KPAD] = jnp.zeros((_BR, 1), jnp.int32)

    e = jnp.where(mask, jnp.exp(sqd * -0.5), 0.0)
    tsum = jnp.sum(e, axis=1, keepdims=True)
    t_ref[...] = e / tsum

    e2 = [jnp.exp(-d) for d in dvals]
    dsum = e2[0]
    for v in e2[1:]:
        dsum = dsum + v
    for k in range(KNN):
        w_ref[:, k:k + 1] = e2[k] / dsum
    w_ref[:, KNN:KPAD] = jnp.zeros((_BR, 1), jnp.float32)


def _knn_transition(pca_pad):
    return pl.pallas_call(
        _knn_body,
        grid=(N // _BR,),
        in_specs=[pl.BlockSpec((N, _PCOLS), lambda i: (0, 0))],
        out_specs=[
            pl.BlockSpec((_BR, N), lambda i: (i, 0)),
            pl.BlockSpec((_BR, KPAD), lambda i: (i, 0)),
            pl.BlockSpec((_BR, KPAD), lambda i: (i, 0)),
        ],
        out_shape=[
            jax.ShapeDtypeStruct((N, N), jnp.float32),
            jax.ShapeDtypeStruct((N, KPAD), jnp.int32),
            jax.ShapeDtypeStruct((N, KPAD), jnp.float32),
        ],
        compiler_params=pltpu.CompilerParams(
            dimension_semantics=("arbitrary",)),
    )(pca_pad)


_RPT = N // 16
_LANES = 16


def _power_body(idx_hbm, w_hbm, d0_hbm, out_hbm, idx_v, w_v, d_v, dn_v, dsh):
    c = lax.axis_index("c")
    s = lax.axis_index("s")
    base = s * _RPT
    pltpu.sync_copy(idx_hbm.at[s], idx_v)
    pltpu.sync_copy(w_hbm.at[s], w_v)
    pltpu.sync_copy(d0_hbm, d_v)

    def step(_, carry):
        for r in range(_RPT // _LANES):
            lo = r * _LANES
            acc = jnp.zeros((_LANES,), jnp.float32)
            for k in range(KPAD):
                ii = idx_v[k, pl.ds(lo, _LANES)]
                acc = acc + w_v[k, pl.ds(lo, _LANES)] * plsc.load_gather(
                    d_v, [ii])
            dn_v[pl.ds(lo, _LANES)] = acc
        pltpu.sync_copy(dn_v, dsh.at[pl.ds(base, _RPT)])
        plsc.subcore_barrier()
        pltpu.sync_copy(dsh, d_v)
        plsc.subcore_barrier()
        return carry

    lax.fori_loop(0, NITER, step, 0)

    mx = jnp.full((_LANES,), -jnp.inf, jnp.float32)
    for j in range(N // _LANES):
        mx = jnp.maximum(mx, d_v[pl.ds(j * _LANES, _LANES)])
    m = lax.reduce_max(mx, (0,))
    for r in range(_RPT // _LANES):
        lo = r * _LANES
        dn_v[pl.ds(lo, _LANES)] = d_v[pl.ds(base + lo, _LANES)] / m

    @pl.when(c == 0)
    def _write():
        pltpu.sync_copy(dn_v, out_hbm.at[pl.ds(base, _RPT)])


def _power_iteration(idx_sc, w_sc, d0):
    mesh = plsc.VectorSubcoreMesh(core_axis_name="c", subcore_axis_name="s")
    kern = functools.partial(
        pl.kernel,
        out_type=jax.ShapeDtypeStruct((N,), jnp.float32),
        mesh=mesh,
        scratch_types=[
            pltpu.VMEM((KPAD, _RPT), jnp.int32),
            pltpu.VMEM((KPAD, _RPT), jnp.float32),
            pltpu.VMEM((N,), jnp.float32),
            pltpu.VMEM((_RPT,), jnp.float32),
            pltpu.VMEM_SHARED((N,), jnp.float32),
        ],
        compiler_params=pltpu.CompilerParams(needs_layout_passes=False),
    )(_power_body)
    return kern(idx_sc, w_sc, d0)


def kernel(expression):
    data = expression
    data_mean = jnp.mean(data, axis=0)
    centered = data - data_mean

    cov = centered.T @ centered / (N - 1)
    eigenvalues, eigenvectors = jnp.linalg.eigh(cov)
    order = jnp.argsort(eigenvalues)[::-1]
    v = eigenvectors[:, order][:, :NCOMP]
    v_pad = jnp.pad(v, ((0, 0), (0, _PCOLS - NCOMP)))

    pca_pad = _project(centered, v_pad)
    pca_data = pca_pad[:, :NCOMP]

    transition, knn_idx, w = _knn_transition(pca_pad)

    idx_sc = knn_idx.reshape(16, _RPT, KPAD).transpose(0, 2, 1)
    w_sc = w.reshape(16, _RPT, KPAD).transpose(0, 2, 1)
    d0 = jnp.zeros((N,), jnp.float32).at[0].set(1.0)
    pseudotime = _power_iteration(idx_sc, w_sc, d0)

    return (pseudotime, transition, pca_data)


# speedup vs baseline: 1.0958x; 1.0024x over previous
import functools

import jax
import jax.numpy as jnp
from jax import lax
from jax.experimental import pallas as pl
from jax.experimental.pallas import tpu as pltpu
from jax.experimental.pallas import tpu_sc as plsc

N = 4096
G = 2048
NCOMP = 50
KNN = 15
KPAD = 16
NITER = 100


_PM = 256
_PCOLS = 64


def _proj_body(x_ref, v_ref, o_ref):
    o_ref[...] = lax.dot_general(
        x_ref[...], v_ref[...], (((1,), (0,)), ((), ())),
        preferred_element_type=jnp.float32,
        precision=lax.Precision.HIGHEST)


def _project(centered, v_pad):
    return pl.pallas_call(
        _proj_body,
        grid=(N // _PM,),
        in_specs=[
            pl.BlockSpec((_PM, G), lambda i: (i, 0)),
            pl.BlockSpec((G, _PCOLS), lambda i: (0, 0)),
        ],
        out_specs=pl.BlockSpec((_PM, _PCOLS), lambda i: (i, 0)),
        out_shape=jax.ShapeDtypeStruct((N, _PCOLS), jnp.float32),
        compiler_params=pltpu.CompilerParams(
            dimension_semantics=("parallel",)),
    )(centered, v_pad)


_BR = 256
_BIG = 3.4e38


def _knn_body(x_ref, t_ref, i_ref, w_ref):
    blk = pl.program_id(0)
    x = x_ref[...]
    xb = x_ref[pl.ds(blk * _BR, _BR), :]
    g = lax.dot_general(xb, x, (((1,), (1,)), ((), ())),
                        preferred_element_type=jnp.float32,
                        precision=lax.Precision.HIGHEST)
    ones = jnp.ones((1, _PCOLS), jnp.float32)
    rsq_j = lax.dot_general(ones, x * x, (((1,), (1,)), ((), ())),
                            preferred_element_type=jnp.float32,
                            precision=lax.Precision.HIGHEST)
    rsq_i = jnp.sum(xb * xb, axis=1, keepdims=True)
    sqd = jnp.maximum(rsq_i + rsq_j - 2.0 * g, 0.0)

    colidx = lax.broadcasted_iota(jnp.int32, (_BR, N), 1)
    bits = lax.bitcast_convert_type(sqd, jnp.int32)
    key = jnp.bitwise_or(jnp.bitwise_and(bits, jnp.int32(-4096)), colidx)
    work = key
    dvals = []
    for k in range(KNN):
        mk = jnp.min(work, axis=1, keepdims=True)
        work = jnp.where(work == mk, jnp.int32(0x7FFFFFFF), work)
        i_ref[:, k:k + 1] = jnp.bitwise_and(mk, jnp.int32(4095))
        dvals.append(lax.bitcast_convert_type(
            jnp.bitwise_and(mk, jnp.int32(-4096)), jnp.float32))
    i_ref[:, KNN:KPAD] = jnp.zeros((_BR, 1), jnp.int32)

    e = jnp.where(work == key, 0.0, jnp.exp(sqd * -0.5))
    tsum = jnp.sum(e, axis=1, keepdims=True)
    t_ref[...] = e / tsum

    e2 = [jnp.exp(-d) for d in dvals]
    dsum = e2[0]
    for v in e2[1:]:
        dsum = dsum + v
    for k in range(KNN):
        w_ref[:, k:k + 1] = e2[k] / dsum
    w_ref[:, KNN:KPAD] = jnp.zeros((_BR, 1), jnp.float32)


def _knn_transition(pca_pad):
    return pl.pallas_call(
        _knn_body,
        grid=(N // _BR,),
        in_specs=[pl.BlockSpec((N, _PCOLS), lambda i: (0, 0))],
        out_specs=[
            pl.BlockSpec((_BR, N), lambda i: (i, 0)),
            pl.BlockSpec((_BR, KPAD), lambda i: (i, 0)),
            pl.BlockSpec((_BR, KPAD), lambda i: (i, 0)),
        ],
        out_shape=[
            jax.ShapeDtypeStruct((N, N), jnp.float32),
            jax.ShapeDtypeStruct((N, KPAD), jnp.int32),
            jax.ShapeDtypeStruct((N, KPAD), jnp.float32),
        ],
        compiler_params=pltpu.CompilerParams(
            dimension_semantics=("arbitrary",)),
    )(pca_pad)


_RPT = N // 16
_LANES = 16


def _power_body(idx_hbm, w_hbm, d0_hbm, out_hbm, idx_v, w_v, d_v, dn_v, dsh):
    c = lax.axis_index("c")
    s = lax.axis_index("s")
    base = s * _RPT
    pltpu.sync_copy(idx_hbm.at[s], idx_v)
    pltpu.sync_copy(w_hbm.at[s], w_v)
    pltpu.sync_copy(d0_hbm, d_v)

    def step(t, carry):
        for r in range(_RPT // _LANES):
            lo = r * _LANES
            acc = jnp.zeros((_LANES,), jnp.float32)
            for k in range(KPAD):
                ii = idx_v[k, pl.ds(lo, _LANES)]
                acc = acc + w_v[k, pl.ds(lo, _LANES)] * plsc.load_gather(
                    d_v, [ii])
            dn_v[pl.ds(lo, _LANES)] = acc
        p = jnp.bitwise_and(t, 1)
        pltpu.sync_copy(dn_v, dsh.at[p, pl.ds(base, _RPT)])
        plsc.subcore_barrier()
        pltpu.sync_copy(dsh.at[p], d_v)
        return carry

    lax.fori_loop(0, NITER, step, 0)

    mx = jnp.full((_LANES,), -jnp.inf, jnp.float32)
    for j in range(N // _LANES):
        mx = jnp.maximum(mx, d_v[pl.ds(j * _LANES, _LANES)])
    m = lax.reduce_max(mx, (0,))
    for r in range(_RPT // _LANES):
        lo = r * _LANES
        dn_v[pl.ds(lo, _LANES)] = d_v[pl.ds(base + lo, _LANES)] / m

    @pl.when(c == 0)
    def _write():
        pltpu.sync_copy(dn_v, out_hbm.at[pl.ds(base, _RPT)])


def _power_iteration(idx_sc, w_sc, d0):
    mesh = plsc.VectorSubcoreMesh(core_axis_name="c", subcore_axis_name="s")
    kern = functools.partial(
        pl.kernel,
        out_type=jax.ShapeDtypeStruct((N,), jnp.float32),
        mesh=mesh,
        scratch_types=[
            pltpu.VMEM((KPAD, _RPT), jnp.int32),
            pltpu.VMEM((KPAD, _RPT), jnp.float32),
            pltpu.VMEM((N,), jnp.float32),
            pltpu.VMEM((_RPT,), jnp.float32),
            pltpu.VMEM_SHARED((2, N), jnp.float32),
        ],
        compiler_params=pltpu.CompilerParams(needs_layout_passes=False),
    )(_power_body)
    return kern(idx_sc, w_sc, d0)


def kernel(expression):
    data = expression
    data_mean = jnp.mean(data, axis=0)
    centered = data - data_mean

    cov = centered.T @ centered / (N - 1)
    eigenvalues, eigenvectors = jnp.linalg.eigh(cov)
    order = jnp.argsort(eigenvalues)[::-1]
    v = eigenvectors[:, order][:, :NCOMP]
    v_pad = jnp.pad(v, ((0, 0), (0, _PCOLS - NCOMP)))

    pca_pad = _project(centered, v_pad)
    pca_data = pca_pad[:, :NCOMP]

    transition, knn_idx, w = _knn_transition(pca_pad)

    idx_sc = knn_idx.reshape(16, _RPT, KPAD).transpose(0, 2, 1)
    w_sc = w.reshape(16, _RPT, KPAD).transpose(0, 2, 1)
    d0 = jnp.zeros((N,), jnp.float32).at[0].set(1.0)
    pseudotime = _power_iteration(idx_sc, w_sc, d0)

    return (pseudotime, transition, pca_data)
